# baseline (speedup 1.0000x reference)
import functools

import jax
import jax.numpy as jnp
from jax import lax
from jax.experimental import pallas as pl
from jax.experimental.pallas import tpu as pltpu, tpu_sc as plsc

EPS = 1e-06
MEAN_READS = 5000.0
LN2 = 0.6931471805599453
SQRT2 = 1.4142135381698608

N_ROWS = 16384
R = 3
NC, NS, L = 2, 16, 16
NW = NC * NS
CHUNK = N_ROWS // NW
VECS = CHUNK // L


def _ln(x):
    bits = lax.bitcast_convert_type(x, jnp.int32)
    e = ((bits >> 23) & 0xFF) - 127
    m = lax.bitcast_convert_type((bits & 0x7FFFFF) | (127 << 23), jnp.float32)
    big = m > SQRT2
    m = jnp.where(big, m * 0.5, m)
    e = e + big.astype(jnp.int32)
    t = (m - 1.0) / (m + 1.0)
    z = t * t
    ln_m = 2.0 * t * (1.0 + z * (1.0 / 3.0 + z * (1.0 / 5.0 + z * (1.0 / 7.0 + z * (1.0 / 9.0)))))
    out = e.astype(jnp.float32) * LN2 + ln_m
    bad = jnp.logical_not(x < jnp.inf)
    return jnp.where(bad, x, out)


def _sc_body(gene_hbm, dsr_hbm, tor_hbm, tt_hbm, out_hbm,
             idx_v, p0_v, p1_v, p2_v, dsr_v, tor_v, sem_in, sem0, sem12,
             sem_out):
    wid = lax.axis_index("s") * NC + lax.axis_index("c")
    base = wid * CHUNK
    H = CHUNK // 2
    d = pltpu.async_copy(dsr_hbm.at[pl.ds(base, CHUNK)], dsr_v, sem_in)
    t = pltpu.async_copy(tor_hbm.at[pl.ds(base, CHUNK)], tor_v, sem_in)
    pltpu.sync_copy(gene_hbm.at[pl.ds(base, H)], idx_v.at[pl.ds(0, H)])
    g0a = pltpu.async_copy(tt_hbm.at[0].at[idx_v.at[pl.ds(0, H)]],
                           p0_v.at[pl.ds(0, H)], sem0)
    pltpu.sync_copy(gene_hbm.at[pl.ds(base + H, H)], idx_v.at[pl.ds(H, H)])
    g0b = pltpu.async_copy(tt_hbm.at[0].at[idx_v.at[pl.ds(H, H)]],
                           p0_v.at[pl.ds(H, H)], sem0)
    g1 = pltpu.async_copy(tt_hbm.at[1].at[idx_v], p1_v, sem12)
    g2 = pltpu.async_copy(tt_hbm.at[2].at[idx_v], p2_v, sem12)
    d.wait()
    t.wait()
    for k in range(VECS):
        s = pl.ds(k * L, L)
        scale = tor_v[s] / (MEAN_READS * dsr_v[s])
        dsr_v[s] = _ln(EPS + scale)
    g0a.wait()
    for k in range(VECS // 2):
        s = pl.ds(k * L, L)
        p0_v[s] = p0_v[s] + dsr_v[s]
    w0a = pltpu.async_copy(p0_v.at[pl.ds(0, H)],
                           out_hbm.at[0].at[pl.ds(base, H)], sem_out)
    g0b.wait()
    for k in range(VECS // 2, VECS):
        s = pl.ds(k * L, L)
        p0_v[s] = p0_v[s] + dsr_v[s]
    w0b = pltpu.async_copy(p0_v.at[pl.ds(H, H)],
                           out_hbm.at[0].at[pl.ds(base + H, H)], sem_out)
    g1.wait()
    g2.wait()
    w1 = pltpu.async_copy(p1_v, out_hbm.at[1].at[pl.ds(base, CHUNK)], sem_out)
    w2 = pltpu.async_copy(p2_v, out_hbm.at[2].at[pl.ds(base, CHUNK)], sem_out)
    w0a.wait()
    w0b.wait()
    w1.wait()
    w2.wait()


@jax.jit
def _sc_call(gene_idx, dsr, tor, table):
    mesh = plsc.VectorSubcoreMesh(core_axis_name="c", subcore_axis_name="s")
    fn = functools.partial(
        pl.kernel,
        out_type=jax.ShapeDtypeStruct((R, N_ROWS), jnp.float32),
        mesh=mesh,
        scratch_types=[
            pltpu.VMEM((CHUNK,), jnp.int32),
            pltpu.VMEM((CHUNK,), jnp.float32),
            pltpu.VMEM((CHUNK,), jnp.float32),
            pltpu.VMEM((CHUNK,), jnp.float32),
            pltpu.VMEM((CHUNK,), jnp.float32),
            pltpu.VMEM((CHUNK,), jnp.float32),
            pltpu.SemaphoreType.DMA,
            pltpu.SemaphoreType.DMA,
            pltpu.SemaphoreType.DMA,
            pltpu.SemaphoreType.DMA,
        ],
        compiler_params=pltpu.CompilerParams(use_tc_tiling_on_sc=False,
                                             needs_layout_passes=False),
    )(_sc_body)
    return fn(gene_idx, dsr, tor, table.T).T


def kernel(gene_index_tensor_n, cell_index_tensor_n, downsampling_rate_tensor_n,
           total_obs_reads_per_cell_tensor_n, cell_features_nf, readout_bias_gr):
    return _sc_call(gene_index_tensor_n.astype(jnp.int32),
                    downsampling_rate_tensor_n,
                    total_obs_reads_per_cell_tensor_n,
                    readout_bias_gr)


# speedup vs baseline: 1.0159x; 1.0159x over previous
import functools

import jax
import jax.numpy as jnp
from jax import lax
from jax.experimental import pallas as pl
from jax.experimental.pallas import tpu as pltpu, tpu_sc as plsc

EPS = 1e-06
MEAN_READS = 5000.0
LN2 = 0.6931471805599453
SQRT2 = 1.4142135381698608

N_ROWS = 16384
R = 3
NC, NS, L = 2, 16, 16
NW = NC * NS
CHUNK = N_ROWS // NW
VECS = CHUNK // L


def _ln(x):
    bits = lax.bitcast_convert_type(x, jnp.int32)
    e = ((bits >> 23) & 0xFF) - 127
    m = lax.bitcast_convert_type((bits & 0x7FFFFF) | (127 << 23), jnp.float32)
    big = m > SQRT2
    m = jnp.where(big, m * 0.5, m)
    e = e + big.astype(jnp.int32)
    t = (m - 1.0) / (m + 1.0)
    z = t * t
    ln_m = 2.0 * t * (1.0 + z * (1.0 / 3.0 + z * (1.0 / 5.0 + z * (1.0 / 7.0 + z * (1.0 / 9.0)))))
    out = e.astype(jnp.float32) * LN2 + ln_m
    bad = jnp.logical_not(x < jnp.inf)
    return jnp.where(bad, x, out)


def _sc_body(gene_hbm, dsr_hbm, tor_hbm, tt_hbm, out_hbm,
             idx_v, p0_v, p1_v, p2_v, dsr_v, tor_v, sem_in, sem0, sem12,
             sem_out):
    wid = lax.axis_index("s") * NC + lax.axis_index("c")
    base = wid * CHUNK
    d = pltpu.async_copy(dsr_hbm.at[pl.ds(base, CHUNK)], dsr_v, sem_in)
    t = pltpu.async_copy(tor_hbm.at[pl.ds(base, CHUNK)], tor_v, sem_in)
    pltpu.sync_copy(gene_hbm.at[pl.ds(base, CHUNK)], idx_v)
    g0 = pltpu.async_copy(tt_hbm.at[0].at[idx_v], p0_v, sem0)
    g1 = pltpu.async_copy(tt_hbm.at[1].at[idx_v], p1_v, sem12)
    g2 = pltpu.async_copy(tt_hbm.at[2].at[idx_v], p2_v, sem12)
    d.wait()
    t.wait()
    for k in range(VECS):
        s = pl.ds(k * L, L)
        scale = tor_v[s] / (MEAN_READS * dsr_v[s])
        dsr_v[s] = _ln(EPS + scale)
    g0.wait()
    for k in range(VECS):
        s = pl.ds(k * L, L)
        p0_v[s] = p0_v[s] + dsr_v[s]
    w0 = pltpu.async_copy(p0_v, out_hbm.at[0].at[pl.ds(base, CHUNK)], sem_out)
    g1.wait()
    g2.wait()
    w1 = pltpu.async_copy(p1_v, out_hbm.at[1].at[pl.ds(base, CHUNK)], sem_out)
    w2 = pltpu.async_copy(p2_v, out_hbm.at[2].at[pl.ds(base, CHUNK)], sem_out)
    w0.wait()
    w1.wait()
    w2.wait()


@jax.jit
def _sc_call(gene_idx, dsr, tor, table):
    mesh = plsc.VectorSubcoreMesh(core_axis_name="c", subcore_axis_name="s")
    fn = functools.partial(
        pl.kernel,
        out_type=jax.ShapeDtypeStruct((R, N_ROWS), jnp.float32),
        mesh=mesh,
        scratch_types=[
            pltpu.VMEM((CHUNK,), jnp.int32),
            pltpu.VMEM((CHUNK,), jnp.float32),
            pltpu.VMEM((CHUNK,), jnp.float32),
            pltpu.VMEM((CHUNK,), jnp.float32),
            pltpu.VMEM((CHUNK,), jnp.float32),
            pltpu.VMEM((CHUNK,), jnp.float32),
            pltpu.SemaphoreType.DMA,
            pltpu.SemaphoreType.DMA,
            pltpu.SemaphoreType.DMA,
            pltpu.SemaphoreType.DMA,
        ],
        compiler_params=pltpu.CompilerParams(use_tc_tiling_on_sc=False,
                                             needs_layout_passes=False),
    )(_sc_body)
    return fn(gene_idx, dsr, tor, table.T).T


def kernel(gene_index_tensor_n, cell_index_tensor_n, downsampling_rate_tensor_n,
           total_obs_reads_per_cell_tensor_n, cell_features_nf, readout_bias_gr):
    return _sc_call(gene_index_tensor_n.astype(jnp.int32),
                    downsampling_rate_tensor_n,
                    total_obs_reads_per_cell_tensor_n,
                    readout_bias_gr)
